# baseline (speedup 1.0000x reference)
import functools

import jax
import jax.numpy as jnp
from jax.experimental import pallas as pl

_NX = 512
_NY = 512
_G = _NX * _NY
_C = 32
_BLK = 2048


def _stage_a(p_ref, pts_ref, seg_ref, fcen_ref):
    p = p_ref[...]
    f32 = jnp.float32
    ptsx = f32(-51.2) + p[0:1, :] * f32(102.4)
    ptsy = f32(-51.2) + p[1:2, :] * f32(102.4)
    ptsz = f32(-3.0) + p[2:3, :] * f32(6.0)
    pts = jnp.concatenate([ptsx, ptsy, ptsz], axis=0)
    pts_ref[...] = pts
    cix = jnp.floor((ptsx - f32(-51.2)) / f32(0.2)).astype(jnp.int32)
    ciy = jnp.floor((ptsy - f32(-51.2)) / f32(0.2)).astype(jnp.int32)
    cx = jnp.clip(cix, 0, _NX - 1)
    cy = jnp.clip(ciy, 0, _NY - 1)
    seg_ref[...] = cx * _NY + cy
    vcx = (cx.astype(jnp.float32) + 0.5) * jnp.float32(0.2) + jnp.float32(-51.2)
    vcy = (cy.astype(jnp.float32) + 0.5) * jnp.float32(0.2) + jnp.float32(-51.2)
    vcz = jnp.zeros_like(vcx)
    fcen_ref[...] = pts - jnp.concatenate([vcx, vcy, vcz], axis=0)


def _stage_b(f_ref, w_ref, b_ref, o_ref):
    acc = jnp.dot(f_ref[...], w_ref[...], preferred_element_type=jnp.float32)
    o_ref[...] = jnp.maximum(acc + b_ref[...], 0.0)


@functools.partial(jax.jit, static_argnames=("n",))
def _point_stage_a(p01, n):
    grid = n // _BLK
    pts, seg, fcen = pl.pallas_call(
        _stage_a,
        grid=(grid,),
        in_specs=[pl.BlockSpec((3, _BLK), lambda i: (0, i))],
        out_specs=[
            pl.BlockSpec((3, _BLK), lambda i: (0, i)),
            pl.BlockSpec((1, _BLK), lambda i: (0, i)),
            pl.BlockSpec((3, _BLK), lambda i: (0, i)),
        ],
        out_shape=[
            jax.ShapeDtypeStruct((3, n), jnp.float32),
            jax.ShapeDtypeStruct((1, n), jnp.int32),
            jax.ShapeDtypeStruct((3, n), jnp.float32),
        ],
    )(p01)
    return pts, seg[0], fcen


@functools.partial(jax.jit, static_argnames=("n",))
def _pfn_layer(feats, W, b, n):
    grid = n // _BLK
    return pl.pallas_call(
        _stage_b,
        grid=(grid,),
        in_specs=[
            pl.BlockSpec((_BLK, 9), lambda i: (i, 0)),
            pl.BlockSpec((9, _C), lambda i: (0, 0)),
            pl.BlockSpec((1, _C), lambda i: (0, 0)),
        ],
        out_specs=pl.BlockSpec((_BLK, _C), lambda i: (i, 0)),
        out_shape=jax.ShapeDtypeStruct((n, _C), jnp.float32),
    )(feats, W, b)


def _frame(points01, W, b, need_voxel_feats):
    n = points01.shape[0]
    pts_t, seg, fcen_t = _point_stage_a(points01.T, n)
    pts = pts_t.T
    ones = jnp.ones((n,), dtype=jnp.float32)
    counts = jax.ops.segment_sum(ones, seg, num_segments=_G)
    ssum = jax.ops.segment_sum(pts, seg, num_segments=_G)
    mean = ssum / jnp.maximum(counts, 1.0)[:, None]
    f_cluster = pts - mean[seg]
    feats = jnp.concatenate([pts, f_cluster, fcen_t.T], axis=1)
    point_feats = _pfn_layer(feats, W, b, n)
    if need_voxel_feats:
        vsum = jax.ops.segment_sum(point_feats, seg, num_segments=_G)
        voxel_feats = vsum / jnp.maximum(counts, 1.0)[:, None]
    else:
        voxel_feats = None
    return voxel_feats, counts, point_feats


def kernel(pc0s_restore, pc1s_restore, pc0s, W, b):
    b2 = b.reshape(1, _C)
    vf0, c0, _ = _frame(pc0s_restore, W, b2, True)
    vf1, c1, _ = _frame(pc1s_restore, W, b2, True)
    _, c2, pf2 = _frame(pc0s, W, b2, False)
    all_voxel_feats_4d = jnp.stack([vf0, vf1], axis=0)
    occupancy = jnp.stack([(c0 > 0), (c1 > 0)], axis=0).astype(jnp.int32)
    pc0_num_voxels = jnp.sum((c2 > 0).astype(jnp.int32))
    return all_voxel_feats_4d, occupancy, pf2, pc0_num_voxels
